# Initial kernel scaffold; baseline (speedup 1.0000x reference)
import jax
import jax.numpy as jnp
from jax.experimental import pallas as pl


def kernel(x, pos, edge_index, batch, We1, We2, Wm1, Wm2, Wu1, Wu2, Wp1, Wp2, Wq1, Wq2):
    raise NotImplementedError("write your pallas kernel here")


# speedup vs baseline: 4.1458x; 4.1458x over previous
import functools

import jax
import jax.numpy as jnp
import numpy as np
from jax import lax
from jax.experimental import pallas as pl
from jax.experimental.pallas import tpu as pltpu
from jax.experimental.pallas import tpu_sc as plsc

_N = 10000
_E = 320000
_NCLS = 16
_H = 128
_A = 4
_L = 4
_G = 64

_NC = 2
_NS = 16
_NW = _NC * _NS
_CH = 128

_mesh = functools.partial(
    plsc.VectorSubcoreMesh, core_axis_name="c", subcore_axis_name="s")
_sc_params = pltpu.CompilerParams(use_tc_tiling_on_sc=False)


def _sc_gather(table, idx, d):
  e = idx.shape[0]
  nch = e // _CH

  @functools.partial(
      pl.kernel,
      out_type=jax.ShapeDtypeStruct((e, d), jnp.float32),
      mesh=_mesh(),
      compiler_params=_sc_params,
      scratch_types=[
          pltpu.VMEM((_CH,), jnp.int32),
          pltpu.VMEM((_CH, d), jnp.float32),
          pltpu.SemaphoreType.DMA,
      ],
  )
  def k(table_hbm, idx_hbm, out_hbm, idx_v, rows_v, sem):
    wid = lax.axis_index("s") * _NC + lax.axis_index("c")
    nit = (nch - wid + _NW - 1) // _NW

    def body(j, carry):
      off = (wid + j * _NW) * _CH
      pltpu.sync_copy(idx_hbm.at[pl.ds(off, _CH)], idx_v)
      pltpu.async_copy(table_hbm.at[idx_v], rows_v, sem).wait()
      pltpu.sync_copy(rows_v, out_hbm.at[pl.ds(off, _CH)])
      return carry

    lax.fori_loop(0, nit, body, 0)

  return k(table, idx)


def _sc_scatter_add(values, idx, d):
  e = values.shape[0]
  nch = e // _CH
  half = _N // 2
  pad = 8
  zrows = (half + pad) // _NS
  wrows = half // 8

  @functools.partial(
      pl.kernel,
      out_type=jax.ShapeDtypeStruct((_N, d), jnp.float32),
      mesh=_mesh(),
      compiler_params=_sc_params,
      scratch_types=[
          pltpu.VMEM((_CH,), jnp.int32),
          pltpu.VMEM((_CH,), jnp.int32),
          pltpu.VMEM((_CH, d), jnp.float32),
          pltpu.VMEM_SHARED((half + pad, d), jnp.float32),
          pltpu.SemaphoreType.DMA,
      ],
  )
  def k(val_hbm, idx_hbm, zeros_hbm, out_hbm, idx_v, idx2_v, val_v, acc_s,
        sem):
    c = lax.axis_index("c")
    s = lax.axis_index("s")
    pltpu.sync_copy(zeros_hbm, acc_s.at[pl.ds(s * zrows, zrows)])
    plsc.subcore_barrier()
    lo = c * half
    nit = (nch - s + _NS - 1) // _NS

    def body(j, carry):
      off = (s + j * _NS) * _CH
      pltpu.sync_copy(idx_hbm.at[pl.ds(off, _CH)], idx_v)
      pltpu.sync_copy(val_hbm.at[pl.ds(off, _CH)], val_v)
      for kk in range(_CH // 16):
        iv = idx_v[pl.ds(kk * 16, 16)]
        t = iv - lo
        oob = (t < 0) | (t >= half)
        idx2_v[pl.ds(kk * 16, 16)] = jnp.where(oob, half, t)
      pltpu.sync_copy(val_v, acc_s.at[idx2_v], add=True)
      return carry

    lax.fori_loop(0, nit, body, 0)
    plsc.subcore_barrier()

    @pl.when(s < 8)
    def _writeout():
      pltpu.sync_copy(acc_s.at[pl.ds(s * wrows, wrows)],
                      out_hbm.at[pl.ds(lo + s * wrows, wrows)])

  zeros = jnp.zeros((zrows, d), jnp.float32)
  return k(values, idx, zeros)


def _silu(x):
  return x * jax.nn.sigmoid(x)


def _tc_geom(ps, pd, t=2000):
  def body(ps_ref, pd_ref, out_ref):
    rel = ps_ref[...] - pd_ref[...]
    r2 = jnp.sum(rel * rel, axis=1, keepdims=True)
    r = jnp.sqrt(r2 + 1e-12)
    u = rel / r
    s3 = np.sqrt(3.0).astype(np.float32)
    ones = jnp.ones_like(r2)
    out_ref[...] = jnp.concatenate(
        [ones, s3 * u[:, 1:2], s3 * u[:, 2:3], s3 * u[:, 0:1], r2, ones,
         jnp.zeros_like(rel[:, 0:10])], axis=1)

  return pl.pallas_call(
      body,
      grid=(_E // t,),
      in_specs=[pl.BlockSpec((t, 16), lambda i: (i, 0)),
                pl.BlockSpec((t, 16), lambda i: (i, 0))],
      out_specs=pl.BlockSpec((t, 16), lambda i: (i, 0)),
      out_shape=jax.ShapeDtypeStruct((_E, 16), jnp.float32),
  )(ps, pd)


def _tc_embed(x2, s16, we1p, we2p, t=2000):
  def body(x_ref, s_ref, we1_ref, we2_ref, h_ref, na_ref):
    sm = s_ref[...]
    cnt = jnp.maximum(sm[:, 5:6], 1.0)
    nar = sm[:, 0:4] / cnt
    na = jnp.concatenate([jnp.ones_like(cnt), nar[:, 1:4]], axis=1)
    oh = (x_ref[...] == lax.broadcasted_iota(jnp.int32, (t, _NCLS), 1))
    oh = oh.astype(jnp.float32)
    f1 = jnp.concatenate([oh * na[:, a:a + 1] for a in range(_A)], axis=1)
    h1 = _silu(jnp.dot(f1, we1_ref[...], preferred_element_type=jnp.float32))
    f2 = jnp.concatenate([h1 * na[:, a:a + 1] for a in range(_A)], axis=1)
    h_ref[...] = jnp.dot(f2, we2_ref[...], preferred_element_type=jnp.float32)
    na_ref[...] = jnp.concatenate([na, jnp.zeros_like(na)], axis=1)

  return pl.pallas_call(
      body,
      grid=(_N // t,),
      in_specs=[
          pl.BlockSpec((t, 1), lambda i: (i, 0)),
          pl.BlockSpec((t, 16), lambda i: (i, 0)),
          pl.BlockSpec((_NCLS * _A, _H), lambda i: (0, 0)),
          pl.BlockSpec((_H * _A, _H), lambda i: (0, 0)),
      ],
      out_specs=[pl.BlockSpec((t, _H), lambda i: (i, 0)),
                 pl.BlockSpec((t, 8), lambda i: (i, 0))],
      out_shape=[jax.ShapeDtypeStruct((_N, _H), jnp.float32),
                 jax.ShapeDtypeStruct((_N, 8), jnp.float32)],
  )(x2, s16, we1p, we2p)


def _tc_edge(hd, hs, g16, w1cat, wdist, w2p, t=2000):
  def body(hd_ref, hs_ref, g_ref, w1_ref, wd_ref, w2_ref, out_ref):
    hd_ = hd_ref[...]
    hs_ = hs_ref[...]
    g = g_ref[...]
    ea = g[:, 0:4]
    dist = g[:, 4:5]
    me = jnp.concatenate(
        [hd_ * ea[:, a:a + 1] for a in range(_A)]
        + [hs_ * ea[:, a:a + 1] for a in range(_A)], axis=1)
    z = jnp.dot(me, w1_ref[...], preferred_element_type=jnp.float32)
    z = z + dist * jnp.dot(ea, wd_ref[...], preferred_element_type=jnp.float32)
    m1 = _silu(z)
    m1e = jnp.concatenate([m1 * ea[:, a:a + 1] for a in range(_A)], axis=1)
    m2 = jnp.dot(m1e, w2_ref[...], preferred_element_type=jnp.float32)
    out_ref[...] = _silu(m2)

  return pl.pallas_call(
      body,
      grid=(_E // t,),
      in_specs=[
          pl.BlockSpec((t, _H), lambda i: (i, 0)),
          pl.BlockSpec((t, _H), lambda i: (i, 0)),
          pl.BlockSpec((t, 16), lambda i: (i, 0)),
          pl.BlockSpec((2 * _H * _A, _H), lambda i: (0, 0)),
          pl.BlockSpec((_A, _H), lambda i: (0, 0)),
          pl.BlockSpec((_H * _A, _H), lambda i: (0, 0)),
      ],
      out_specs=pl.BlockSpec((t, _H), lambda i: (i, 0)),
      out_shape=jax.ShapeDtypeStruct((_E, _H), jnp.float32),
  )(hd, hs, g16, w1cat, wdist, w2p)


def _tc_update(h, s2, na8, wu1p, wu2p, t=2000):
  def body(h_ref, s_ref, na_ref, w1_ref, w2_ref, out_ref):
    h_ = h_ref[...]
    agg = s_ref[...]
    na = na_ref[:, 0:4]
    cat = jnp.concatenate([h_, agg], axis=1)
    f1 = jnp.concatenate([cat * na[:, a:a + 1] for a in range(_A)], axis=1)
    u1 = _silu(jnp.dot(f1, w1_ref[...], preferred_element_type=jnp.float32))
    f2 = jnp.concatenate([u1 * na[:, a:a + 1] for a in range(_A)], axis=1)
    u2 = jnp.dot(f2, w2_ref[...], preferred_element_type=jnp.float32)
    out_ref[...] = h_ + u2

  return pl.pallas_call(
      body,
      grid=(_N // t,),
      in_specs=[
          pl.BlockSpec((t, _H), lambda i: (i, 0)),
          pl.BlockSpec((t, _H), lambda i: (i, 0)),
          pl.BlockSpec((t, 8), lambda i: (i, 0)),
          pl.BlockSpec((2 * _H * _A, _H), lambda i: (0, 0)),
          pl.BlockSpec((_H * _A, _H), lambda i: (0, 0)),
      ],
      out_specs=pl.BlockSpec((t, _H), lambda i: (i, 0)),
      out_shape=jax.ShapeDtypeStruct((_N, _H), jnp.float32),
  )(h, s2, na8, wu1p, wu2p)


def _tc_head(h, na8, b2, wp1p, wp2, t=2000):
  def body(h_ref, na_ref, b_ref, w1_ref, w2_ref, pooled_ref, cnt_ref):
    i = pl.program_id(0)
    na = na_ref[:, 0:4]
    h_ = h_ref[...]
    f1 = jnp.concatenate([h_ * na[:, a:a + 1] for a in range(_A)], axis=1)
    p1 = _silu(jnp.dot(f1, w1_ref[...], preferred_element_type=jnp.float32))
    p2 = jnp.dot(p1, w2_ref[...], preferred_element_type=jnp.float32)
    ohb = (b_ref[...] == lax.broadcasted_iota(jnp.int32, (t, _G), 1))
    ohb = ohb.astype(jnp.float32)
    pp = jax.lax.dot_general(ohb, p2, (((0,), (0,)), ((), ())),
                             preferred_element_type=jnp.float32)
    cc = jnp.sum(ohb, axis=0, keepdims=True)

    @pl.when(i == 0)
    def _init():
      pooled_ref[...] = pp
      cnt_ref[...] = cc

    @pl.when(i > 0)
    def _acc():
      pooled_ref[...] += pp
      cnt_ref[...] += cc

  return pl.pallas_call(
      body,
      grid=(_N // t,),
      in_specs=[
          pl.BlockSpec((t, _H), lambda i: (i, 0)),
          pl.BlockSpec((t, 8), lambda i: (i, 0)),
          pl.BlockSpec((t, 1), lambda i: (i, 0)),
          pl.BlockSpec((_H * _A, _H), lambda i: (0, 0)),
          pl.BlockSpec((_H, _H), lambda i: (0, 0)),
      ],
      out_specs=[pl.BlockSpec((_G, _H), lambda i: (0, 0)),
                 pl.BlockSpec((1, _G), lambda i: (0, 0))],
      out_shape=[jax.ShapeDtypeStruct((_G, _H), jnp.float32),
                 jax.ShapeDtypeStruct((1, _G), jnp.float32)],
  )(h, na8, b2, wp1p, wp2)


def _tc_final(pooled, cnt, wq1, wq2):
  def body(p_ref, c_ref, w1_ref, w2_ref, o_ref):
    cnt_col = jnp.maximum(c_ref[...], 1.0).reshape(_G, 1)
    p = p_ref[...] / cnt_col
    z = _silu(jnp.dot(p, w1_ref[...], preferred_element_type=jnp.float32))
    o_ref[...] = jnp.dot(z, w2_ref[...], preferred_element_type=jnp.float32)

  return pl.pallas_call(
      body,
      out_shape=jax.ShapeDtypeStruct((_G, 1), jnp.float32),
  )(pooled, cnt, wq1, wq2)


def _perm(w, f):
  return w.reshape(f, _A, -1).transpose(1, 0, 2).reshape(f * _A, -1)


def kernel(x, pos, edge_index, batch, We1, We2, Wm1, Wm2, Wu1, Wu2, Wp1, Wp2,
           Wq1, Wq2):
  src = edge_index[0]
  dst = edge_index[1]

  we1p = _perm(We1, _NCLS)
  we2p = _perm(We2, _H)
  wp1p = _perm(Wp1, _H)

  pos_pad = jnp.zeros((_N, 16), jnp.float32).at[:, 0:3].set(pos)
  x2 = x.reshape(_N, 1)
  b2 = batch.reshape(_N, 1)

  ps = _sc_gather(pos_pad, src, 16)
  pd = _sc_gather(pos_pad, dst, 16)
  g16 = _tc_geom(ps, pd)

  s16 = _sc_scatter_add(g16, dst, 16)
  h, na8 = _tc_embed(x2, s16, we1p, we2p)

  for i in range(_L):
    w1r = Wm1[i].reshape(2 * _H + 1, _A, _H)
    w1cat = jnp.concatenate([
        w1r[0:_H].transpose(1, 0, 2).reshape(_H * _A, _H),
        w1r[_H:2 * _H].transpose(1, 0, 2).reshape(_H * _A, _H)], axis=0)
    wdist = w1r[2 * _H]
    w2p = _perm(Wm2[i], _H)
    wu1p = _perm(Wu1[i], 2 * _H)
    wu2p = _perm(Wu2[i], _H)

    hd = _sc_gather(h, dst, _H)
    hs = _sc_gather(h, src, _H)
    m2 = _tc_edge(hd, hs, g16, w1cat, wdist, w2p)
    s2 = _sc_scatter_add(m2, dst, _H)
    h = _tc_update(h, s2, na8, wu1p, wu2p)

  pooled, cnt = _tc_head(h, na8, b2, wp1p, Wp2)
  return _tc_final(pooled, cnt, Wq1, Wq2)


# speedup vs baseline: 5.2097x; 1.2566x over previous
import functools

import jax
import jax.numpy as jnp
import numpy as np
from jax import lax
from jax.experimental import pallas as pl
from jax.experimental.pallas import tpu as pltpu
from jax.experimental.pallas import tpu_sc as plsc

_N = 10000
_E = 320000
_NCLS = 16
_H = 128
_A = 4
_L = 4
_G = 64

_NC = 2
_NS = 16
_NW = _NC * _NS
_CHG = 100
_CHS = 80

_mesh = functools.partial(
    plsc.VectorSubcoreMesh, core_axis_name="c", subcore_axis_name="s")
_sc_params = pltpu.CompilerParams(use_tc_tiling_on_sc=False)


def _sc_gather2(table, idxa, idxb, d):
  e = idxa.shape[0]
  epw = e // _NW
  nch = epw // _CHG

  @functools.partial(
      pl.kernel,
      out_type=(jax.ShapeDtypeStruct((e, d), jnp.float32),
                jax.ShapeDtypeStruct((e, d), jnp.float32)),
      mesh=_mesh(),
      compiler_params=_sc_params,
      scratch_types=[
          pltpu.VMEM((nch, _CHG), jnp.int32),
          pltpu.VMEM((nch, _CHG), jnp.int32),
          pltpu.VMEM((2, _CHG, d), jnp.float32),
          pltpu.VMEM((2, _CHG, d), jnp.float32),
          pltpu.SemaphoreType.DMA,
      ],
  )
  def k(table_hbm, ia_hbm, ib_hbm, oa_hbm, ob_hbm, ia_v, ib_v, ra_v, rb_v,
        sem):
    wid = lax.axis_index("s") * _NC + lax.axis_index("c")
    base = wid * epw
    pltpu.sync_copy(ia_hbm.at[wid], ia_v)
    pltpu.sync_copy(ib_hbm.at[wid], ib_v)

    def issue(t):
      slot = lax.rem(t, 2)
      pltpu.async_copy(table_hbm.at[ia_v.at[t]], ra_v.at[slot], sem)
      pltpu.async_copy(table_hbm.at[ib_v.at[t]], rb_v.at[slot], sem)

    def drain(t):
      slot = lax.rem(t, 2)
      off = base + t * _CHG
      pltpu.make_async_copy(table_hbm.at[ia_v.at[t]], ra_v.at[slot],
                            sem).wait()
      pltpu.make_async_copy(table_hbm.at[ib_v.at[t]], rb_v.at[slot],
                            sem).wait()
      pltpu.sync_copy(ra_v.at[slot], oa_hbm.at[pl.ds(off, _CHG)])
      pltpu.sync_copy(rb_v.at[slot], ob_hbm.at[pl.ds(off, _CHG)])

    issue(0)

    def body(t, carry):
      issue(t + 1)
      drain(t)
      return carry

    lax.fori_loop(0, nch - 1, body, 0)
    drain(nch - 1)

  ia = idxa.reshape(_NW, nch, _CHG)
  ib = idxb.reshape(_NW, nch, _CHG)
  return k(table, ia, ib)


def _sc_scatter_add(values, idx, d):
  e = values.shape[0]
  ept = e // _NS
  nch = ept // _CHS
  half = _N // 2
  pad = 24
  zrows = (half + pad) // _NS
  wrows = half // 8

  @functools.partial(
      pl.kernel,
      out_type=jax.ShapeDtypeStruct((_N, d), jnp.float32),
      mesh=_mesh(),
      compiler_params=_sc_params,
      scratch_types=[
          pltpu.VMEM((nch, _CHS), jnp.int32),
          pltpu.VMEM((2, _CHS), jnp.int32),
          pltpu.VMEM((2, _CHS, d), jnp.float32),
          pltpu.VMEM_SHARED((half + pad, d), jnp.float32),
          pltpu.SemaphoreType.DMA,
      ],
  )
  def k(val_hbm, idx_hbm, zeros_hbm, out_hbm, idx_v, idx2_v, val_v, acc_s,
        sem):
    c = lax.axis_index("c")
    s = lax.axis_index("s")
    pltpu.sync_copy(idx_hbm.at[s], idx_v)
    pltpu.sync_copy(zeros_hbm, acc_s.at[pl.ds(s * zrows, zrows)])
    plsc.subcore_barrier()
    lo = c * half
    base = s * ept

    def issue(t):
      slot = lax.rem(t, 2)
      off = base + t * _CHS
      pltpu.async_copy(val_hbm.at[pl.ds(off, _CHS)], val_v.at[slot], sem)

    def consume(t):
      slot = lax.rem(t, 2)
      off = base + t * _CHS
      for kk in range(_CHS // 16):
        iv = idx_v[t, pl.ds(kk * 16, 16)]
        tt = iv - lo
        oob = (tt < 0) | (tt >= half)
        idx2_v[slot, pl.ds(kk * 16, 16)] = jnp.where(oob, half, tt)
      pltpu.make_async_copy(val_hbm.at[pl.ds(off, _CHS)], val_v.at[slot],
                            sem).wait()
      pltpu.sync_copy(val_v.at[slot], acc_s.at[idx2_v.at[slot]], add=True)

    issue(0)

    def body(t, carry):
      issue(t + 1)
      consume(t)
      return carry

    lax.fori_loop(0, nch - 1, body, 0)
    consume(nch - 1)
    plsc.subcore_barrier()

    @pl.when(s < 8)
    def _writeout():
      pltpu.sync_copy(acc_s.at[pl.ds(s * wrows, wrows)],
                      out_hbm.at[pl.ds(lo + s * wrows, wrows)])

  zeros = jnp.zeros((zrows, d), jnp.float32)
  idx2d = idx.reshape(_NS, nch, _CHS)
  return k(values, idx2d, zeros)


def _silu(x):
  return x * jax.nn.sigmoid(x)


def _tc_geom(ps, pd, t=2000):
  def body(ps_ref, pd_ref, out_ref):
    rel = ps_ref[...] - pd_ref[...]
    r2 = jnp.sum(rel * rel, axis=1, keepdims=True)
    r = jnp.sqrt(r2 + 1e-12)
    u = rel / r
    s3 = np.sqrt(3.0).astype(np.float32)
    ones = jnp.ones_like(r2)
    out_ref[...] = jnp.concatenate(
        [ones, s3 * u[:, 1:2], s3 * u[:, 2:3], s3 * u[:, 0:1], r2, ones,
         jnp.zeros_like(rel[:, 0:10])], axis=1)

  return pl.pallas_call(
      body,
      grid=(_E // t,),
      in_specs=[pl.BlockSpec((t, 16), lambda i: (i, 0)),
                pl.BlockSpec((t, 16), lambda i: (i, 0))],
      out_specs=pl.BlockSpec((t, 16), lambda i: (i, 0)),
      out_shape=jax.ShapeDtypeStruct((_E, 16), jnp.float32),
  )(ps, pd)


def _tc_embed(x2, s16, we1p, we2p, t=2000):
  def body(x_ref, s_ref, we1_ref, we2_ref, h_ref, na_ref):
    sm = s_ref[...]
    cnt = jnp.maximum(sm[:, 5:6], 1.0)
    nar = sm[:, 0:4] / cnt
    na = jnp.concatenate([jnp.ones_like(cnt), nar[:, 1:4]], axis=1)
    oh = (x_ref[...] == lax.broadcasted_iota(jnp.int32, (t, _NCLS), 1))
    oh = oh.astype(jnp.float32)
    f1 = jnp.concatenate([oh * na[:, a:a + 1] for a in range(_A)], axis=1)
    h1 = _silu(jnp.dot(f1, we1_ref[...], preferred_element_type=jnp.float32))
    f2 = jnp.concatenate([h1 * na[:, a:a + 1] for a in range(_A)], axis=1)
    h_ref[...] = jnp.dot(f2, we2_ref[...], preferred_element_type=jnp.float32)
    na_ref[...] = jnp.concatenate([na, jnp.zeros_like(na)], axis=1)

  return pl.pallas_call(
      body,
      grid=(_N // t,),
      in_specs=[
          pl.BlockSpec((t, 1), lambda i: (i, 0)),
          pl.BlockSpec((t, 16), lambda i: (i, 0)),
          pl.BlockSpec((_NCLS * _A, _H), lambda i: (0, 0)),
          pl.BlockSpec((_H * _A, _H), lambda i: (0, 0)),
      ],
      out_specs=[pl.BlockSpec((t, _H), lambda i: (i, 0)),
                 pl.BlockSpec((t, 8), lambda i: (i, 0))],
      out_shape=[jax.ShapeDtypeStruct((_N, _H), jnp.float32),
                 jax.ShapeDtypeStruct((_N, 8), jnp.float32)],
  )(x2, s16, we1p, we2p)


def _tc_edge(hd, hs, g16, w1cat, wdist, w2p, t=2000):
  def body(hd_ref, hs_ref, g_ref, w1_ref, wd_ref, w2_ref, out_ref):
    hd_ = hd_ref[...]
    hs_ = hs_ref[...]
    g = g_ref[...]
    ea = g[:, 0:4]
    dist = g[:, 4:5]
    me = jnp.concatenate(
        [hd_ * ea[:, a:a + 1] for a in range(_A)]
        + [hs_ * ea[:, a:a + 1] for a in range(_A)], axis=1)
    z = jnp.dot(me, w1_ref[...], preferred_element_type=jnp.float32)
    z = z + jnp.dot(dist * ea, wd_ref[...], preferred_element_type=jnp.float32)
    m1 = _silu(z)
    m1e = jnp.concatenate([m1 * ea[:, a:a + 1] for a in range(_A)], axis=1)
    m2 = jnp.dot(m1e, w2_ref[...], preferred_element_type=jnp.float32)
    out_ref[...] = _silu(m2)

  return pl.pallas_call(
      body,
      grid=(_E // t,),
      in_specs=[
          pl.BlockSpec((t, _H), lambda i: (i, 0)),
          pl.BlockSpec((t, _H), lambda i: (i, 0)),
          pl.BlockSpec((t, 16), lambda i: (i, 0)),
          pl.BlockSpec((2 * _H * _A, _H), lambda i: (0, 0)),
          pl.BlockSpec((_A, _H), lambda i: (0, 0)),
          pl.BlockSpec((_H * _A, _H), lambda i: (0, 0)),
      ],
      out_specs=pl.BlockSpec((t, _H), lambda i: (i, 0)),
      out_shape=jax.ShapeDtypeStruct((_E, _H), jnp.float32),
  )(hd, hs, g16, w1cat, wdist, w2p)


def _tc_update(h, s2, na8, wu1p, wu2p, t=2000):
  def body(h_ref, s_ref, na_ref, w1_ref, w2_ref, out_ref):
    h_ = h_ref[...]
    agg = s_ref[...]
    na = na_ref[:, 0:4]
    cat = jnp.concatenate([h_, agg], axis=1)
    f1 = jnp.concatenate([cat * na[:, a:a + 1] for a in range(_A)], axis=1)
    u1 = _silu(jnp.dot(f1, w1_ref[...], preferred_element_type=jnp.float32))
    f2 = jnp.concatenate([u1 * na[:, a:a + 1] for a in range(_A)], axis=1)
    u2 = jnp.dot(f2, w2_ref[...], preferred_element_type=jnp.float32)
    out_ref[...] = h_ + u2

  return pl.pallas_call(
      body,
      grid=(_N // t,),
      in_specs=[
          pl.BlockSpec((t, _H), lambda i: (i, 0)),
          pl.BlockSpec((t, _H), lambda i: (i, 0)),
          pl.BlockSpec((t, 8), lambda i: (i, 0)),
          pl.BlockSpec((2 * _H * _A, _H), lambda i: (0, 0)),
          pl.BlockSpec((_H * _A, _H), lambda i: (0, 0)),
      ],
      out_specs=pl.BlockSpec((t, _H), lambda i: (i, 0)),
      out_shape=jax.ShapeDtypeStruct((_N, _H), jnp.float32),
  )(h, s2, na8, wu1p, wu2p)


def _tc_head(h, na8, b2, wp1p, wp2, t=2000):
  def body(h_ref, na_ref, b_ref, w1_ref, w2_ref, pooled_ref, cnt_ref):
    i = pl.program_id(0)
    na = na_ref[:, 0:4]
    h_ = h_ref[...]
    f1 = jnp.concatenate([h_ * na[:, a:a + 1] for a in range(_A)], axis=1)
    p1 = _silu(jnp.dot(f1, w1_ref[...], preferred_element_type=jnp.float32))
    p2 = jnp.dot(p1, w2_ref[...], preferred_element_type=jnp.float32)
    ohb = (b_ref[...] == lax.broadcasted_iota(jnp.int32, (t, _G), 1))
    ohb = ohb.astype(jnp.float32)
    pp = jax.lax.dot_general(ohb, p2, (((0,), (0,)), ((), ())),
                             preferred_element_type=jnp.float32)
    cc = jnp.sum(ohb, axis=0, keepdims=True)

    @pl.when(i == 0)
    def _init():
      pooled_ref[...] = pp
      cnt_ref[...] = cc

    @pl.when(i > 0)
    def _acc():
      pooled_ref[...] += pp
      cnt_ref[...] += cc

  return pl.pallas_call(
      body,
      grid=(_N // t,),
      in_specs=[
          pl.BlockSpec((t, _H), lambda i: (i, 0)),
          pl.BlockSpec((t, 8), lambda i: (i, 0)),
          pl.BlockSpec((t, 1), lambda i: (i, 0)),
          pl.BlockSpec((_H * _A, _H), lambda i: (0, 0)),
          pl.BlockSpec((_H, _H), lambda i: (0, 0)),
      ],
      out_specs=[pl.BlockSpec((_G, _H), lambda i: (0, 0)),
                 pl.BlockSpec((1, _G), lambda i: (0, 0))],
      out_shape=[jax.ShapeDtypeStruct((_G, _H), jnp.float32),
                 jax.ShapeDtypeStruct((1, _G), jnp.float32)],
  )(h, na8, b2, wp1p, wp2)


def _tc_final(pooled, cnt, wq1, wq2):
  def body(p_ref, c_ref, w1_ref, w2_ref, o_ref):
    cnt_col = jnp.maximum(c_ref[...], 1.0).reshape(_G, 1)
    p = p_ref[...] / cnt_col
    z = _silu(jnp.dot(p, w1_ref[...], preferred_element_type=jnp.float32))
    o_ref[...] = jnp.dot(z, w2_ref[...], preferred_element_type=jnp.float32)

  return pl.pallas_call(
      body,
      out_shape=jax.ShapeDtypeStruct((_G, 1), jnp.float32),
  )(pooled, cnt, wq1, wq2)


def _perm(w, f):
  return w.reshape(f, _A, -1).transpose(1, 0, 2).reshape(f * _A, -1)


def kernel(x, pos, edge_index, batch, We1, We2, Wm1, Wm2, Wu1, Wu2, Wp1, Wp2,
           Wq1, Wq2):
  src = edge_index[0]
  dst = edge_index[1]

  we1p = _perm(We1, _NCLS)
  we2p = _perm(We2, _H)
  wp1p = _perm(Wp1, _H)

  pos_pad = jnp.zeros((_N, 16), jnp.float32).at[:, 0:3].set(pos)
  x2 = x.reshape(_N, 1)
  b2 = batch.reshape(_N, 1)

  ps, pd = _sc_gather2(pos_pad, src, dst, 16)
  g16 = _tc_geom(ps, pd)

  s16 = _sc_scatter_add(g16, dst, 16)
  h, na8 = _tc_embed(x2, s16, we1p, we2p)

  for i in range(_L):
    w1r = Wm1[i].reshape(2 * _H + 1, _A, _H)
    w1cat = jnp.concatenate([
        w1r[0:_H].transpose(1, 0, 2).reshape(_H * _A, _H),
        w1r[_H:2 * _H].transpose(1, 0, 2).reshape(_H * _A, _H)], axis=0)
    wdist = w1r[2 * _H]
    w2p = _perm(Wm2[i], _H)
    wu1p = _perm(Wu1[i], 2 * _H)
    wu2p = _perm(Wu2[i], _H)

    hd, hs = _sc_gather2(h, dst, src, _H)
    m2 = _tc_edge(hd, hs, g16, w1cat, wdist, w2p)
    s2 = _sc_scatter_add(m2, dst, _H)
    h = _tc_update(h, s2, na8, wu1p, wu2p)

  pooled, cnt = _tc_head(h, na8, b2, wp1p, Wp2)
  return _tc_final(pooled, cnt, Wq1, Wq2)
